# baseline (speedup 1.0000x reference)
import functools

import jax
import jax.numpy as jnp
from jax import lax
from jax.experimental import pallas as pl
from jax.experimental.pallas import tpu as pltpu
from jax.experimental.pallas import tpu_sc as plsc

_B = 16384
_L = 200
_LP = _L // 2
_V = 256
_NC = 2
_NS = 16
_NW = _NC * _NS
_NH = 4
_BH = _B // _NH
_RPW = _BH // _NW
_NG = _RPW // 4


def _sc_hist(packed3):
  mesh = plsc.VectorSubcoreMesh(core_axis_name="c", subcore_axis_name="s")

  @functools.partial(
      pl.kernel,
      out_type=jax.ShapeDtypeStruct((_NW * _NG, _V), jnp.int32),
      mesh=mesh,
      compiler_params=pltpu.CompilerParams(use_tc_tiling_on_sc=False,
                                           needs_layout_passes=False),
      scratch_types=[
          pltpu.VMEM((_LP, _RPW), jnp.int32),
          pltpu.VMEM((_NG, _V), jnp.int32),
      ],
  )
  def k(pk_hbm, out_hbm, pk_v, hist):
    wid = lax.axis_index("s") * _NC + lax.axis_index("c")
    iota = lax.iota(jnp.int32, 16)
    zero16 = jnp.zeros((16,), jnp.int32)

    def zrow(q, carry):
      for j in range(_V // 16):
        hist[q, pl.ds(j * 16, 16)] = zero16
      return carry

    lax.fori_loop(0, _NG, zrow, 0)

    pltpu.sync_copy(pk_hbm.at[wid], pk_v)

    def tok(t, carry):
      for ki in range(4):
        for g in range(_NG // 16):
          w = pk_v[t, pl.ds(ki * _NG + g * 16, 16)]
          id0 = w & 255
          id1 = (w >> 8) & 255
          sh0 = 16 - 8 * ki
          v0 = (w & 0x10000) >> sh0 if sh0 >= 0 else (w & 0x10000) << (-sh0)
          sh1 = 24 - 8 * ki
          v1 = (w & 0x1000000) >> sh1 if sh1 > 0 else (w & 0x1000000)
          qv = iota + (g * 16)
          plsc.addupdate_scatter(hist, [qv, id0], v0)
          plsc.addupdate_scatter(hist, [qv, id1], v1)
      return carry

    lax.fori_loop(0, _LP, tok, 0)
    pltpu.sync_copy(hist, out_hbm.at[pl.ds(wid * _NG, _NG)])

  return k(packed3)


_HG = 8


def _tc_head(hp, emb_ext, w1, b1, w2, b2):
  hb = _NW * _NG // _HG

  def body(h_ref, emb_ref, w1_ref, b1_ref, w2_ref, b2_ref, out_ref):
    h = h_ref[...].reshape(hb // _NG, _NG, _V)
    planes = [((h >> (8 * k)) & 255).astype(jnp.float32) for k in range(4)]
    counts = jnp.concatenate(planes, axis=1).reshape(4 * hb, _V)
    sums = jnp.dot(counts, emb_ref[...],
                   preferred_element_type=jnp.float32)
    cnt = jnp.sum(counts, axis=1, keepdims=True)
    pooled = sums / jnp.maximum(cnt, 1.0)
    proj = jnp.maximum(
        jnp.dot(pooled, w1_ref[...],
                preferred_element_type=jnp.float32) + b1_ref[...], 0.0)
    o = jnp.dot(proj, w2_ref[...],
                preferred_element_type=jnp.float32) + b2_ref[...]
    out_ref[...] = o[:, :6]

  return pl.pallas_call(
      body,
      grid=(_HG,),
      in_specs=[
          pl.BlockSpec((hb, _V), lambda b: (b, 0)),
          pl.BlockSpec((_V, 128), lambda b: (0, 0)),
          pl.BlockSpec((128, 128), lambda b: (0, 0)),
          pl.BlockSpec((1, 128), lambda b: (0, 0)),
          pl.BlockSpec((128, 128), lambda b: (0, 0)),
          pl.BlockSpec((1, 128), lambda b: (0, 0)),
      ],
      out_specs=pl.BlockSpec((4 * hb, 6), lambda b: (b, 0)),
      out_shape=jax.ShapeDtypeStruct((_BH, 6), jnp.float32),
  )(hp, emb_ext, w1, b1, w2, b2)


def kernel(input_ids, attention_mask, token_type_ids, emb, W_proj, b_proj,
           W_head, b_head):
  del token_type_ids
  ids = input_ids.astype(jnp.int32)
  mask = attention_mask.astype(jnp.int32)
  emb_ext = jnp.zeros((_V, 128), jnp.float32).at[:, :12].set(emb)
  emb_ext = emb_ext.at[:, 12].set(1.0)
  w1 = jnp.zeros((128, 128), jnp.float32).at[:12, :12].set(W_proj.T)
  b1 = jnp.zeros((1, 128), jnp.float32).at[0, :12].set(b_proj)
  w2 = jnp.zeros((128, 128), jnp.float32).at[:12, :6].set(W_head.T)
  b2 = jnp.zeros((1, 128), jnp.float32).at[0, :6].set(b_head)

  outs = []
  for h in range(_NH):
    idh = lax.slice_in_dim(ids, h * _BH, (h + 1) * _BH, axis=0)
    mkh = lax.slice_in_dim(mask, h * _BH, (h + 1) * _BH, axis=0)
    packed = (idh[:, :_LP] | (idh[:, _LP:] << 8)
              | (mkh[:, :_LP] << 16) | (mkh[:, _LP:] << 24))
    packed3 = packed.reshape(_NW, _RPW, _LP).transpose(0, 2, 1)
    hp = _sc_hist(packed3)
    outs.append(_tc_head(hp, emb_ext, w1, b1, w2, b2))
  return jnp.concatenate(outs, axis=0)


# speedup vs baseline: 1.1095x; 1.1095x over previous
import functools

import jax
import jax.numpy as jnp
from jax import lax
from jax.experimental import pallas as pl
from jax.experimental.pallas import tpu as pltpu
from jax.experimental.pallas import tpu_sc as plsc

_B = 16384
_L = 200
_LP = _L // 2
_V = 256
_NC = 2
_NS = 16
_NW = _NC * _NS
_NH = 2
_BH = _B // _NH
_RPW = _BH // _NW
_NG = _RPW // 4


def _sc_hist(packed3):
  mesh = plsc.VectorSubcoreMesh(core_axis_name="c", subcore_axis_name="s")

  @functools.partial(
      pl.kernel,
      out_type=jax.ShapeDtypeStruct((_NW * _NG, _V), jnp.int32),
      mesh=mesh,
      compiler_params=pltpu.CompilerParams(use_tc_tiling_on_sc=False,
                                           needs_layout_passes=False),
      scratch_types=[
          pltpu.VMEM((_LP, _RPW), jnp.int32),
          pltpu.VMEM((_NG, _V), jnp.int32),
      ],
  )
  def k(pk_hbm, out_hbm, pk_v, hist):
    wid = lax.axis_index("s") * _NC + lax.axis_index("c")
    iota = lax.iota(jnp.int32, 16)
    zero16 = jnp.zeros((16,), jnp.int32)

    def zrow(q, carry):
      for j in range(_V // 16):
        hist[q, pl.ds(j * 16, 16)] = zero16
      return carry

    lax.fori_loop(0, _NG, zrow, 0)

    pltpu.sync_copy(pk_hbm.at[wid], pk_v)

    def tok(t, carry):
      for ki in range(4):
        for g in range(_NG // 16):
          w = pk_v[t, pl.ds(ki * _NG + g * 16, 16)]
          id0 = w & 255
          id1 = (w >> 8) & 255
          sh0 = 16 - 8 * ki
          v0 = (w & 0x10000) >> sh0 if sh0 >= 0 else (w & 0x10000) << (-sh0)
          sh1 = 24 - 8 * ki
          v1 = (w & 0x1000000) >> sh1 if sh1 > 0 else (w & 0x1000000)
          qv = iota + (g * 16)
          plsc.addupdate_scatter(hist, [qv, id0], v0)
          plsc.addupdate_scatter(hist, [qv, id1], v1)
      return carry

    lax.fori_loop(0, _LP, tok, 0)
    pltpu.sync_copy(hist, out_hbm.at[pl.ds(wid * _NG, _NG)])

  return k(packed3)


_HG = 8


def _tc_head(hp, emb_ext, w1, b1, w2, b2):
  hb = _NW * _NG // _HG

  def body(h_ref, emb_ref, w1_ref, b1_ref, w2_ref, b2_ref, out_ref):
    h = h_ref[...].reshape(hb // _NG, _NG, _V)
    planes = [((h >> (8 * k)) & 255).astype(jnp.float32) for k in range(4)]
    counts = jnp.concatenate(planes, axis=1).reshape(4 * hb, _V)
    sums = jnp.dot(counts, emb_ref[...],
                   preferred_element_type=jnp.float32)
    cnt = jnp.sum(counts, axis=1, keepdims=True)
    pooled = sums / jnp.maximum(cnt, 1.0)
    proj = jnp.maximum(
        jnp.dot(pooled, w1_ref[...],
                preferred_element_type=jnp.float32) + b1_ref[...], 0.0)
    o = jnp.dot(proj, w2_ref[...],
                preferred_element_type=jnp.float32) + b2_ref[...]
    out_ref[...] = o[:, :6]

  return pl.pallas_call(
      body,
      grid=(_HG,),
      in_specs=[
          pl.BlockSpec((hb, _V), lambda b: (b, 0)),
          pl.BlockSpec((_V, 128), lambda b: (0, 0)),
          pl.BlockSpec((128, 128), lambda b: (0, 0)),
          pl.BlockSpec((1, 128), lambda b: (0, 0)),
          pl.BlockSpec((128, 128), lambda b: (0, 0)),
          pl.BlockSpec((1, 128), lambda b: (0, 0)),
      ],
      out_specs=pl.BlockSpec((4 * hb, 6), lambda b: (b, 0)),
      out_shape=jax.ShapeDtypeStruct((_BH, 6), jnp.float32),
  )(hp, emb_ext, w1, b1, w2, b2)


def kernel(input_ids, attention_mask, token_type_ids, emb, W_proj, b_proj,
           W_head, b_head):
  del token_type_ids
  ids = input_ids.astype(jnp.int32)
  mask = attention_mask.astype(jnp.int32)
  emb_ext = jnp.zeros((_V, 128), jnp.float32).at[:, :12].set(emb)
  emb_ext = emb_ext.at[:, 12].set(1.0)
  w1 = jnp.zeros((128, 128), jnp.float32).at[:12, :12].set(W_proj.T)
  b1 = jnp.zeros((1, 128), jnp.float32).at[0, :12].set(b_proj)
  w2 = jnp.zeros((128, 128), jnp.float32).at[:12, :6].set(W_head.T)
  b2 = jnp.zeros((1, 128), jnp.float32).at[0, :6].set(b_head)

  outs = []
  for h in range(_NH):
    idh = lax.slice_in_dim(ids, h * _BH, (h + 1) * _BH, axis=0)
    mkh = lax.slice_in_dim(mask, h * _BH, (h + 1) * _BH, axis=0)
    packed = (idh[:, :_LP] | (idh[:, _LP:] << 8)
              | (mkh[:, :_LP] << 16) | (mkh[:, _LP:] << 24))
    packed3 = packed.reshape(_NW, _RPW, _LP).transpose(0, 2, 1)
    hp = _sc_hist(packed3)
    outs.append(_tc_head(hp, emb_ext, w1, b1, w2, b2))
  return jnp.concatenate(outs, axis=0)
